# Initial kernel scaffold; baseline (speedup 1.0000x reference)
import jax
import jax.numpy as jnp
from jax.experimental import pallas as pl


def kernel(x_author, x_paper, dst_ap, src_ap, val_ap, dst_pa, src_pa, val_pa, dst_pp, src_pp, val_pp, d_ap, d_pa, d_pp, W1_a, b1_a, W1_p, b1_p, W2, b2, lw):
    raise NotImplementedError("write your pallas kernel here")


# speedup vs baseline: 1.3054x; 1.3054x over previous
import jax
import jax.numpy as jnp
from jax import lax
from jax.experimental import pallas as pl
from jax.experimental.pallas import tpu as pltpu
from jax.experimental.pallas import tpu_sc as plsc

N_A = 50000
N_P = 100000
D_IN = 128
HID = 64
OUT = 16
HOP = 3
E = 500000

FW = 16
NQ = HID // FW
NC = 2
NS = 16
G = 128
E_PAD = 500224
NA_PAD = 50048
NP_PAD = 100352


def _make_spmm(n_groups, has_lam, n_out_pad, table_pads):
    rpt = n_out_pad // NS
    n_win = E_PAD // G
    q_per_core = NQ // NC

    def body(*refs):
        g_refs = []
        i = 0
        for _ in range(n_groups):
            step = 5 if has_lam else 4
            g_refs.append(refs[i:i + step])
            i += step
        init_ref, out_ref = refs[i], refs[i + 1]
        (src_v, dstb, val_v, gidx, rowbuf, lamv, acc, sem) = refs[i + 2:]

        cid = lax.axis_index("c")
        sid = lax.axis_index("s")

        if has_lam:
            for gi in range(n_groups):
                pltpu.sync_copy(g_refs[gi][4], lamv.at[gi])

        e16 = [jnp.full((16,), e, jnp.int32) for e in range(16)]

        for qi in range(q_per_core):
            q = cid * q_per_core + qi
            off_r = sid * rpt
            pltpu.sync_copy(init_ref.at[pl.ds(q * n_out_pad + off_r, rpt)],
                            acc.at[pl.ds(off_r, rpt)])
            plsc.subcore_barrier()

            for gi in range(n_groups):
                table_ref, src_ref, dst_ref, val_ref = g_refs[gi][:4]
                qb16 = jnp.full((16,), q * table_pads[gi], jnp.int32)
                if has_lam:
                    lamvec = lamv[gi, pl.ds(0, 16)]

                def wbody(t, _, table_ref=table_ref, src_ref=src_ref,
                          dst_ref=dst_ref, val_ref=val_ref, qb16=qb16,
                          lamvec=(lamvec if has_lam else None)):
                    base = (sid + t * NS) * G
                    pltpu.sync_copy(src_ref.at[pl.ds(base, G)], src_v)
                    pltpu.sync_copy(dst_ref.at[pl.ds(base, G)], dstb)
                    pltpu.sync_copy(val_ref.at[pl.ds(base, G)], val_v)
                    for gs in range(G // 16):
                        sl = pl.ds(gs * 16, 16)
                        gidx[sl] = src_v[sl] + qb16
                    pltpu.async_copy(table_ref.at[gidx], rowbuf, sem).wait()
                    for gs in range(G // 16):
                        vals16 = val_v[pl.ds(gs * 16, 16)]
                        if lamvec is not None:
                            vals16 = vals16 * lamvec
                        for e in range(16):
                            vb = jnp.take(vals16, e16[e])
                            r = gs * 16 + e
                            rowbuf[r, pl.ds(0, 16)] = (
                                rowbuf[r, pl.ds(0, 16)] * vb)
                    pltpu.sync_copy(rowbuf, acc.at[dstb], add=True)
                    return 0

                n_t = (n_win - sid + NS - 1) // NS
                lax.fori_loop(0, n_t, wbody, 0)

            plsc.subcore_barrier()
            pltpu.sync_copy(acc.at[pl.ds(off_r, rpt)],
                            out_ref.at[pl.ds(q * n_out_pad + off_r, rpt)])
            if qi + 1 < q_per_core:
                plsc.subcore_barrier()

    mesh = plsc.VectorSubcoreMesh(core_axis_name="c", subcore_axis_name="s")
    scratch = [
        pltpu.VMEM((G,), jnp.int32),
        pltpu.VMEM((G,), jnp.int32),
        pltpu.VMEM((G,), jnp.float32),
        pltpu.VMEM((G,), jnp.int32),
        pltpu.VMEM((G, FW), jnp.float32),
        pltpu.VMEM((max(n_groups, 1), 16), jnp.float32),
        pltpu.VMEM_SHARED((n_out_pad, FW), jnp.float32),
        pltpu.SemaphoreType.DMA,
    ]
    return pl.kernel(
        body,
        out_type=jax.ShapeDtypeStruct((NQ * n_out_pad, FW), jnp.float32),
        mesh=mesh,
        scratch_types=scratch,
        compiler_params=pltpu.CompilerParams(use_tc_tiling_on_sc=False),
    )


_spmm_a = _make_spmm(n_groups=1, has_lam=False, n_out_pad=NA_PAD,
                     table_pads=[NP_PAD])
_spmm_p = _make_spmm(n_groups=2, has_lam=True, n_out_pad=NP_PAD,
                     table_pads=[NA_PAD, NP_PAD])


def _fc1_author(x, w, b, d):
    br = 2000

    def body(x_ref, w_ref, b_ref, d_ref, xa_ref, ba_ref):
        h = jnp.dot(x_ref[...], w_ref[...],
                    preferred_element_type=jnp.float32) + b_ref[...]
        h = jnp.maximum(h, 0.0)
        dh = d_ref[...] * h
        for q in range(NQ):
            xa_ref[q] = h[:, q * FW:(q + 1) * FW]
            ba_ref[q] = dh[:, q * FW:(q + 1) * FW]

    return pl.pallas_call(
        body,
        grid=(N_A // br,),
        in_specs=[
            pl.BlockSpec((br, D_IN), lambda i: (i, 0)),
            pl.BlockSpec((D_IN, HID), lambda i: (0, 0)),
            pl.BlockSpec((1, HID), lambda i: (0, 0)),
            pl.BlockSpec((br, 1), lambda i: (i, 0)),
        ],
        out_specs=[
            pl.BlockSpec((NQ, br, FW), lambda i: (0, i, 0)),
            pl.BlockSpec((NQ, br, FW), lambda i: (0, i, 0)),
        ],
        out_shape=[
            jax.ShapeDtypeStruct((NQ, NA_PAD, FW), jnp.float32),
            jax.ShapeDtypeStruct((NQ, NA_PAD, FW), jnp.float32),
        ],
    )(x, w, b.reshape(1, HID), d)


def _fc1_paper(x, w, b, d1, d2, lam):
    br = 2000

    def body(x_ref, w_ref, b_ref, d1_ref, d2_ref, lam_ref, xp_ref, ini_ref):
        h = jnp.dot(x_ref[...], w_ref[...],
                    preferred_element_type=jnp.float32) + b_ref[...]
        h = jnp.maximum(h, 0.0)
        u = d1_ref[...] * h
        v = d2_ref[...] * h
        lamm = lam_ref[...]
        for q in range(NQ):
            xp_ref[q] = h[:, q * FW:(q + 1) * FW]
        for hh in range(HOP):
            ini = lamm[hh, 0] * u + lamm[hh, 1] * v
            for q in range(NQ):
                ini_ref[hh, q] = ini[:, q * FW:(q + 1) * FW]

    return pl.pallas_call(
        body,
        grid=(N_P // br,),
        in_specs=[
            pl.BlockSpec((br, D_IN), lambda i: (i, 0)),
            pl.BlockSpec((D_IN, HID), lambda i: (0, 0)),
            pl.BlockSpec((1, HID), lambda i: (0, 0)),
            pl.BlockSpec((br, 1), lambda i: (i, 0)),
            pl.BlockSpec((br, 1), lambda i: (i, 0)),
            pl.BlockSpec((HOP, 2), lambda i: (0, 0)),
        ],
        out_specs=[
            pl.BlockSpec((NQ, br, FW), lambda i: (0, i, 0)),
            pl.BlockSpec((HOP, NQ, br, FW), lambda i: (0, 0, i, 0)),
        ],
        out_shape=[
            jax.ShapeDtypeStruct((NQ, NP_PAD, FW), jnp.float32),
            jax.ShapeDtypeStruct((HOP, NQ, NP_PAD, FW), jnp.float32),
        ],
    )(x, w, b.reshape(1, HID), d1, d2, lam)


def _fc2(hq, w, b):
    br = 2000

    def body(h_ref, w_ref, b_ref, o_ref):
        h = jnp.concatenate([h_ref[q] for q in range(NQ)], axis=1)
        o_ref[...] = jnp.dot(h, w_ref[...],
                             preferred_element_type=jnp.float32) + b_ref[...]

    return pl.pallas_call(
        body,
        grid=(N_P // br,),
        in_specs=[
            pl.BlockSpec((NQ, br, FW), lambda i: (0, i, 0)),
            pl.BlockSpec((HID, OUT), lambda i: (0, 0)),
            pl.BlockSpec((1, OUT), lambda i: (0, 0)),
        ],
        out_specs=pl.BlockSpec((br, OUT), lambda i: (i, 0)),
        out_shape=jax.ShapeDtypeStruct((N_P, OUT), jnp.float32),
    )(hq, w, b.reshape(1, OUT))


def _pad_edges(src, dst, val):
    npad = E_PAD - E
    sp = (jnp.arange(npad, dtype=jnp.int32) % 61)
    dp = (jnp.arange(npad, dtype=jnp.int32) % 53)
    vp = jnp.zeros((npad,), jnp.float32)
    return (jnp.concatenate([src.astype(jnp.int32), sp]),
            jnp.concatenate([dst.astype(jnp.int32), dp]),
            jnp.concatenate([val, vp]))


def kernel(x_author, x_paper, dst_ap, src_ap, val_ap, dst_pa, src_pa,
           val_pa, dst_pp, src_pp, val_pp, d_ap, d_pa, d_pp, W1_a, b1_a,
           W1_p, b1_p, W2, b2, lw):
    lam_p = jax.nn.softmax(lw[:, 1:3], axis=-1)
    lam16 = jnp.broadcast_to(lam_p[:, :, None], (HOP, 2, 16))

    s_ap, d_ap_e, v_ap = _pad_edges(src_ap, dst_ap, val_ap)
    s_pa, d_pa_e, v_pa = _pad_edges(src_pa, dst_pa, val_pa)
    s_pp, d_pp_e, v_pp = _pad_edges(src_pp, dst_pp, val_pp)

    xa_q, base_a = _fc1_author(x_author, W1_a, b1_a, d_ap)
    xp_q, init_p = _fc1_paper(x_paper, W1_p, b1_p, d_pa, d_pp, lam_p)

    base_a = base_a.reshape(NQ * NA_PAD, FW)
    init_p = init_p.reshape(HOP, NQ * NP_PAD, FW)

    hp = xp_q.reshape(NQ * NP_PAD, FW)
    for i in range(HOP):
        ha = _spmm_a(hp, s_ap, d_ap_e, v_ap, base_a)
        hp = _spmm_p(ha, s_pa, d_pa_e, v_pa, lam16[i, 0],
                     hp, s_pp, d_pp_e, v_pp, lam16[i, 1],
                     init_p[i])
    return _fc2(hp.reshape(NQ, NP_PAD, FW), W2, b2)


# speedup vs baseline: 3.3871x; 2.5946x over previous
import jax
import jax.numpy as jnp
from jax import lax
from jax.experimental import pallas as pl
from jax.experimental.pallas import tpu as pltpu
from jax.experimental.pallas import tpu_sc as plsc

N_A = 50000
N_P = 100000
D_IN = 128
HID = 64
OUT = 16
HOP = 3
E = 500000

FW = 16
NQ = HID // FW
NC = 2
NS = 16
G = 128
WPT = 246
E_PAD = NS * G * WPT
NA_PAD = 50048
NP_PAD = 100352


def _make_spmm(n_groups, has_lam, n_out_pad, table_pads):
    rpt = n_out_pad // NS
    n_win = E_PAD // G
    q_per_core = NQ // NC

    def body(*refs):
        g_refs = []
        i = 0
        for _ in range(n_groups):
            step = 5 if has_lam else 4
            g_refs.append(refs[i:i + step])
            i += step
        init_ref, out_ref = refs[i], refs[i + 1]
        (src_v, dstb, sdst, val_v, gidx, rowbuf, lamv, acc,
         si0, si1, sg0, sg1, ss0, ss1) = refs[i + 2:]
        s_idx, s_gat, s_sca = (si0, si1), (sg0, sg1), (ss0, ss1)

        cid = lax.axis_index("c")
        sid = lax.axis_index("s")

        if has_lam:
            for gi in range(n_groups):
                pltpu.sync_copy(g_refs[gi][4], lamv.at[gi])

        e16 = [jnp.full((16,), e, jnp.int32) for e in range(16)]

        for qi in range(q_per_core):
            q = cid * q_per_core + qi
            off_r = sid * rpt
            pltpu.sync_copy(init_ref.at[pl.ds(q * n_out_pad + off_r, rpt)],
                            acc.at[pl.ds(off_r, rpt)])
            plsc.subcore_barrier()

            for gi in range(n_groups):
                table_ref, src_ref, dst_ref, val_ref = g_refs[gi][:4]
                qb16 = jnp.full((16,), q * table_pads[gi], jnp.int32)
                lamvec = lamv[gi, pl.ds(0, 16)] if has_lam else None

                def fire_idx(t, b, src_ref=src_ref, dst_ref=dst_ref,
                             val_ref=val_ref):
                    tc = jnp.minimum(t, WPT - 1)
                    base = (sid + tc * NS) * G
                    pltpu.async_copy(src_ref.at[pl.ds(base, G)],
                                     src_v.at[b], s_idx[b])
                    pltpu.async_copy(dst_ref.at[pl.ds(base, G)],
                                     dstb.at[b], s_idx[b])
                    pltpu.async_copy(val_ref.at[pl.ds(base, G)],
                                     val_v.at[b], s_idx[b])

                def wait_idx(b, src_ref=src_ref, dst_ref=dst_ref,
                             val_ref=val_ref):
                    pltpu.make_async_copy(src_ref.at[pl.ds(0, G)],
                                          src_v.at[b], s_idx[b]).wait()
                    pltpu.make_async_copy(dst_ref.at[pl.ds(0, G)],
                                          dstb.at[b], s_idx[b]).wait()
                    pltpu.make_async_copy(val_ref.at[pl.ds(0, G)],
                                          val_v.at[b], s_idx[b]).wait()

                def comp_gidx(b):
                    for gs in range(G // 16):
                        sl = pl.ds(gs * 16, 16)
                        gidx[b, sl] = src_v[b, sl] + qb16

                def fire_gather(b, table_ref=table_ref):
                    pltpu.async_copy(table_ref.at[gidx.at[b]],
                                     rowbuf.at[b], s_gat[b])

                def wait_gather(b, table_ref=table_ref):
                    pltpu.make_async_copy(table_ref.at[gidx.at[b]],
                                          rowbuf.at[b], s_gat[b]).wait()

                def wait_scatter(b):
                    pltpu.make_async_copy(rowbuf.at[b],
                                          acc.at[sdst.at[b]],
                                          s_sca[b]).wait()

                def emit_half(t, b, lamvec=lamvec):
                    nb = 1 - b
                    wait_idx(nb)
                    comp_gidx(nb)
                    fire_gather(nb)
                    wait_gather(b)

                    @pl.when(t > 0)
                    def _():
                        wait_scatter(nb)

                    for gs in range(G // 16):
                        sl = pl.ds(gs * 16, 16)
                        sdst[b, sl] = dstb[b, sl]
                        vals16 = val_v[b, sl]
                        if lamvec is not None:
                            vals16 = vals16 * lamvec
                        for e in range(16):
                            vb = jnp.take(vals16, e16[e])
                            r = gs * 16 + e
                            rowbuf[b, r, pl.ds(0, 16)] = (
                                rowbuf[b, r, pl.ds(0, 16)] * vb)
                    pltpu.async_copy(rowbuf.at[b], acc.at[sdst.at[b]],
                                     s_sca[b], add=True)
                    fire_idx(t + 2, b)

                fire_idx(jnp.int32(0), 0)
                fire_idx(jnp.int32(1), 1)
                wait_idx(0)
                comp_gidx(0)
                fire_gather(0)

                def pair(tt, _):
                    emit_half(tt * 2, 0)
                    emit_half(tt * 2 + 1, 1)
                    return 0

                lax.fori_loop(0, WPT // 2, pair, 0)

                wait_idx(1)
                wait_gather(0)
                wait_scatter(1)

            plsc.subcore_barrier()
            pltpu.sync_copy(acc.at[pl.ds(off_r, rpt)],
                            out_ref.at[pl.ds(q * n_out_pad + off_r, rpt)])
            if qi + 1 < q_per_core:
                plsc.subcore_barrier()

    mesh = plsc.VectorSubcoreMesh(core_axis_name="c", subcore_axis_name="s")
    scratch = [
        pltpu.VMEM((2, G), jnp.int32),
        pltpu.VMEM((2, G), jnp.int32),
        pltpu.VMEM((2, G), jnp.int32),
        pltpu.VMEM((2, G), jnp.float32),
        pltpu.VMEM((2, G), jnp.int32),
        pltpu.VMEM((2, G, FW), jnp.float32),
        pltpu.VMEM((max(n_groups, 1), 16), jnp.float32),
        pltpu.VMEM_SHARED((n_out_pad, FW), jnp.float32),
        pltpu.SemaphoreType.DMA,
        pltpu.SemaphoreType.DMA,
        pltpu.SemaphoreType.DMA,
        pltpu.SemaphoreType.DMA,
        pltpu.SemaphoreType.DMA,
        pltpu.SemaphoreType.DMA,
    ]
    return pl.kernel(
        body,
        out_type=jax.ShapeDtypeStruct((NQ * n_out_pad, FW), jnp.float32),
        mesh=mesh,
        scratch_types=scratch,
        compiler_params=pltpu.CompilerParams(use_tc_tiling_on_sc=False),
    )


_spmm_a = _make_spmm(n_groups=1, has_lam=False, n_out_pad=NA_PAD,
                     table_pads=[NP_PAD])
_spmm_p = _make_spmm(n_groups=2, has_lam=True, n_out_pad=NP_PAD,
                     table_pads=[NA_PAD, NP_PAD])


def _fc1_author(x, w, b, d):
    br = 2000

    def body(x_ref, w_ref, b_ref, d_ref, xa_ref, ba_ref):
        h = jnp.dot(x_ref[...], w_ref[...],
                    preferred_element_type=jnp.float32) + b_ref[...]
        h = jnp.maximum(h, 0.0)
        dh = d_ref[...] * h
        for q in range(NQ):
            xa_ref[q] = h[:, q * FW:(q + 1) * FW]
            ba_ref[q] = dh[:, q * FW:(q + 1) * FW]

    return pl.pallas_call(
        body,
        grid=(N_A // br,),
        in_specs=[
            pl.BlockSpec((br, D_IN), lambda i: (i, 0)),
            pl.BlockSpec((D_IN, HID), lambda i: (0, 0)),
            pl.BlockSpec((1, HID), lambda i: (0, 0)),
            pl.BlockSpec((br, 1), lambda i: (i, 0)),
        ],
        out_specs=[
            pl.BlockSpec((NQ, br, FW), lambda i: (0, i, 0)),
            pl.BlockSpec((NQ, br, FW), lambda i: (0, i, 0)),
        ],
        out_shape=[
            jax.ShapeDtypeStruct((NQ, NA_PAD, FW), jnp.float32),
            jax.ShapeDtypeStruct((NQ, NA_PAD, FW), jnp.float32),
        ],
    )(x, w, b.reshape(1, HID), d)


def _fc1_paper(x, w, b, d1, d2, lam):
    br = 2000

    def body(x_ref, w_ref, b_ref, d1_ref, d2_ref, lam_ref, xp_ref, ini_ref):
        h = jnp.dot(x_ref[...], w_ref[...],
                    preferred_element_type=jnp.float32) + b_ref[...]
        h = jnp.maximum(h, 0.0)
        u = d1_ref[...] * h
        v = d2_ref[...] * h
        lamm = lam_ref[...]
        for q in range(NQ):
            xp_ref[q] = h[:, q * FW:(q + 1) * FW]
        for hh in range(HOP):
            ini = lamm[hh, 0] * u + lamm[hh, 1] * v
            for q in range(NQ):
                ini_ref[hh, q] = ini[:, q * FW:(q + 1) * FW]

    return pl.pallas_call(
        body,
        grid=(N_P // br,),
        in_specs=[
            pl.BlockSpec((br, D_IN), lambda i: (i, 0)),
            pl.BlockSpec((D_IN, HID), lambda i: (0, 0)),
            pl.BlockSpec((1, HID), lambda i: (0, 0)),
            pl.BlockSpec((br, 1), lambda i: (i, 0)),
            pl.BlockSpec((br, 1), lambda i: (i, 0)),
            pl.BlockSpec((HOP, 2), lambda i: (0, 0)),
        ],
        out_specs=[
            pl.BlockSpec((NQ, br, FW), lambda i: (0, i, 0)),
            pl.BlockSpec((HOP, NQ, br, FW), lambda i: (0, 0, i, 0)),
        ],
        out_shape=[
            jax.ShapeDtypeStruct((NQ, NP_PAD, FW), jnp.float32),
            jax.ShapeDtypeStruct((HOP, NQ, NP_PAD, FW), jnp.float32),
        ],
    )(x, w, b.reshape(1, HID), d1, d2, lam)


def _fc2(hq, w, b):
    br = 2000

    def body(h_ref, w_ref, b_ref, o_ref):
        h = jnp.concatenate([h_ref[q] for q in range(NQ)], axis=1)
        o_ref[...] = jnp.dot(h, w_ref[...],
                             preferred_element_type=jnp.float32) + b_ref[...]

    return pl.pallas_call(
        body,
        grid=(N_P // br,),
        in_specs=[
            pl.BlockSpec((NQ, br, FW), lambda i: (0, i, 0)),
            pl.BlockSpec((HID, OUT), lambda i: (0, 0)),
            pl.BlockSpec((1, OUT), lambda i: (0, 0)),
        ],
        out_specs=pl.BlockSpec((br, OUT), lambda i: (i, 0)),
        out_shape=jax.ShapeDtypeStruct((N_P, OUT), jnp.float32),
    )(hq, w, b.reshape(1, OUT))


def _pad_edges(src, dst, val):
    npad = E_PAD - E
    sp = (jnp.arange(npad, dtype=jnp.int32) % 61)
    dp = (jnp.arange(npad, dtype=jnp.int32) % 53)
    vp = jnp.zeros((npad,), jnp.float32)
    return (jnp.concatenate([src.astype(jnp.int32), sp]),
            jnp.concatenate([dst.astype(jnp.int32), dp]),
            jnp.concatenate([val, vp]))


def kernel(x_author, x_paper, dst_ap, src_ap, val_ap, dst_pa, src_pa,
           val_pa, dst_pp, src_pp, val_pp, d_ap, d_pa, d_pp, W1_a, b1_a,
           W1_p, b1_p, W2, b2, lw):
    lam_p = jax.nn.softmax(lw[:, 1:3], axis=-1)
    lam16 = jnp.broadcast_to(lam_p[:, :, None], (HOP, 2, 16))

    s_ap, d_ap_e, v_ap = _pad_edges(src_ap, dst_ap, val_ap)
    s_pa, d_pa_e, v_pa = _pad_edges(src_pa, dst_pa, val_pa)
    s_pp, d_pp_e, v_pp = _pad_edges(src_pp, dst_pp, val_pp)

    xa_q, base_a = _fc1_author(x_author, W1_a, b1_a, d_ap)
    xp_q, init_p = _fc1_paper(x_paper, W1_p, b1_p, d_pa, d_pp, lam_p)

    base_a = base_a.reshape(NQ * NA_PAD, FW)
    init_p = init_p.reshape(HOP, NQ * NP_PAD, FW)

    hp = xp_q.reshape(NQ * NP_PAD, FW)
    for i in range(HOP):
        ha = _spmm_a(hp, s_ap, d_ap_e, v_ap, base_a)
        hp = _spmm_p(ha, s_pa, d_pa_e, v_pa, lam16[i, 0],
                     hp, s_pp, d_pp_e, v_pp, lam16[i, 1],
                     init_p[i])
    return _fc2(hp.reshape(NQ, NP_PAD, FW), W2, b2)


# speedup vs baseline: 4.5051x; 1.3301x over previous
import jax
import jax.numpy as jnp
from jax import lax
from jax.experimental import pallas as pl
from jax.experimental.pallas import tpu as pltpu
from jax.experimental.pallas import tpu_sc as plsc

N_A = 50000
N_P = 100000
D_IN = 128
HID = 64
OUT = 16
HOP = 3
E = 500000

FW = 16
NQ = HID // FW
NC = 2
NS = 16
G = 256
E_PAD_P = 503808
WPT_P = 2 * E_PAD_P // (NS * G)
E_PAD_A = 507904
WPT_A = E_PAD_A // (NS * G)
WB_P = E_PAD_P // G
NA_PAD = 50048
NP_PAD = 100352


def _make_spmm(two_tables, n_out_pad, pad1, pad2, wpt, wb):
    rpt = n_out_pad // NS
    q_per_core = NQ // NC

    def body(*refs):
        if two_tables:
            (t1_ref, t2_ref, src_ref, dst_ref, val_ref, lam1_ref, lam2_ref,
             init_ref, out_ref, src_v, dstb, sdst, val_v, gidx, rowbuf,
             lamv, acc, si0, si1, sg0, sg1, ss0, ss1) = refs
        else:
            (t1_ref, src_ref, dst_ref, val_ref, init_ref, out_ref,
             src_v, dstb, sdst, val_v, gidx, rowbuf,
             lamv, acc, si0, si1, sg0, sg1, ss0, ss1) = refs
            t2_ref = None
        s_idx, s_gat, s_sca = (si0, si1), (sg0, sg1), (ss0, ss1)

        cid = lax.axis_index("c")
        sid = lax.axis_index("s")

        if two_tables:
            pltpu.sync_copy(lam1_ref, lamv.at[0])
            pltpu.sync_copy(lam2_ref, lamv.at[1])

        e16 = [jnp.full((16,), e, jnp.int32) for e in range(16)]

        def fire_idx(t, b):
            tc = jnp.minimum(t, wpt - 1)
            base = (sid + tc * NS) * G
            pltpu.async_copy(src_ref.at[pl.ds(base, G)],
                             src_v.at[b], s_idx[b])
            pltpu.async_copy(dst_ref.at[pl.ds(base, G)],
                             dstb.at[b], s_idx[b])
            pltpu.async_copy(val_ref.at[pl.ds(base, G)],
                             val_v.at[b], s_idx[b])

        def wait_idx(b):
            pltpu.make_async_copy(src_ref.at[pl.ds(0, G)],
                                  src_v.at[b], s_idx[b]).wait()
            pltpu.make_async_copy(dst_ref.at[pl.ds(0, G)],
                                  dstb.at[b], s_idx[b]).wait()
            pltpu.make_async_copy(val_ref.at[pl.ds(0, G)],
                                  val_v.at[b], s_idx[b]).wait()

        def wait_gather(b):
            pltpu.make_async_copy(t1_ref.at[gidx.at[b]],
                                  rowbuf.at[b], s_gat[b]).wait()

        def wait_scatter(b):
            pltpu.make_async_copy(rowbuf.at[b], acc.at[sdst.at[b]],
                                  s_sca[b]).wait()

        for qi in range(q_per_core):
            q = cid * q_per_core + qi
            off_r = sid * rpt
            pltpu.sync_copy(init_ref.at[pl.ds(q * n_out_pad + off_r, rpt)],
                            acc.at[pl.ds(off_r, rpt)])
            plsc.subcore_barrier()

            qb1 = jnp.full((16,), q * pad1, jnp.int32)
            if two_tables:
                qb2 = jnp.full((16,), q * pad2, jnp.int32)

            def comp_gidx_fire(t, b):
                tc = jnp.minimum(t, wpt - 1)
                w = sid + tc * NS
                if two_tables:
                    is2 = w >= wb
                    qb = jnp.where(is2, qb2, qb1)
                else:
                    qb = qb1
                for gs in range(G // 16):
                    sl = pl.ds(gs * 16, 16)
                    gidx[b, sl] = src_v[b, sl] + qb
                if two_tables:
                    @pl.when(is2)
                    def _():
                        pltpu.async_copy(t2_ref.at[gidx.at[b]],
                                         rowbuf.at[b], s_gat[b])

                    @pl.when(jnp.logical_not(is2))
                    def _():
                        pltpu.async_copy(t1_ref.at[gidx.at[b]],
                                         rowbuf.at[b], s_gat[b])
                else:
                    pltpu.async_copy(t1_ref.at[gidx.at[b]],
                                     rowbuf.at[b], s_gat[b])

            def emit_half(t, b):
                nb = 1 - b
                wait_idx(nb)
                comp_gidx_fire(t + 1, nb)
                wait_gather(b)

                @pl.when(t > 0)
                def _():
                    wait_scatter(nb)

                if two_tables:
                    w = sid + t * NS
                    lsel = (w >= wb).astype(jnp.float32)
                    lamvec = (lamv[1, pl.ds(0, 16)] * lsel
                              + lamv[0, pl.ds(0, 16)] * (1.0 - lsel))
                else:
                    lamvec = None
                for gs in range(G // 16):
                    sl = pl.ds(gs * 16, 16)
                    sdst[b, sl] = dstb[b, sl]
                    vals16 = val_v[b, sl]
                    if lamvec is not None:
                        vals16 = vals16 * lamvec
                    for e in range(16):
                        vb = jnp.take(vals16, e16[e])
                        r = gs * 16 + e
                        rowbuf[b, r, pl.ds(0, 16)] = (
                            rowbuf[b, r, pl.ds(0, 16)] * vb)
                pltpu.async_copy(rowbuf.at[b], acc.at[sdst.at[b]],
                                 s_sca[b], add=True)
                fire_idx(t + 2, b)

            fire_idx(jnp.int32(0), 0)
            fire_idx(jnp.int32(1), 1)
            wait_idx(0)
            comp_gidx_fire(jnp.int32(0), 0)

            def pair(tt, _):
                emit_half(tt * 2, 0)
                emit_half(tt * 2 + 1, 1)
                return 0

            lax.fori_loop(0, wpt // 2, pair, 0)

            wait_idx(1)
            wait_gather(0)
            wait_scatter(1)

            plsc.subcore_barrier()
            pltpu.sync_copy(acc.at[pl.ds(off_r, rpt)],
                            out_ref.at[pl.ds(q * n_out_pad + off_r, rpt)])
            if qi + 1 < q_per_core:
                plsc.subcore_barrier()

    mesh = plsc.VectorSubcoreMesh(core_axis_name="c", subcore_axis_name="s")
    scratch = [
        pltpu.VMEM((2, G), jnp.int32),
        pltpu.VMEM((2, G), jnp.int32),
        pltpu.VMEM((2, G), jnp.int32),
        pltpu.VMEM((2, G), jnp.float32),
        pltpu.VMEM((2, G), jnp.int32),
        pltpu.VMEM((2, G, FW), jnp.float32),
        pltpu.VMEM((2, 16), jnp.float32),
        pltpu.VMEM_SHARED((n_out_pad, FW), jnp.float32),
        pltpu.SemaphoreType.DMA,
        pltpu.SemaphoreType.DMA,
        pltpu.SemaphoreType.DMA,
        pltpu.SemaphoreType.DMA,
        pltpu.SemaphoreType.DMA,
        pltpu.SemaphoreType.DMA,
    ]
    return pl.kernel(
        body,
        out_type=jax.ShapeDtypeStruct((NQ * n_out_pad, FW), jnp.float32),
        mesh=mesh,
        scratch_types=scratch,
        compiler_params=pltpu.CompilerParams(use_tc_tiling_on_sc=False),
    )


_spmm_a = _make_spmm(two_tables=False, n_out_pad=NA_PAD, pad1=NP_PAD,
                     pad2=0, wpt=WPT_A, wb=0)
_spmm_p = _make_spmm(two_tables=True, n_out_pad=NP_PAD, pad1=NA_PAD,
                     pad2=NP_PAD, wpt=WPT_P, wb=WB_P)


def _fc1_author(x, w, b, d):
    br = 2000

    def body(x_ref, w_ref, b_ref, d_ref, xa_ref, ba_ref):
        h = jnp.dot(x_ref[...], w_ref[...],
                    preferred_element_type=jnp.float32) + b_ref[...]
        h = jnp.maximum(h, 0.0)
        dh = d_ref[...] * h
        for q in range(NQ):
            xa_ref[q] = h[:, q * FW:(q + 1) * FW]
            ba_ref[q] = dh[:, q * FW:(q + 1) * FW]

    return pl.pallas_call(
        body,
        grid=(N_A // br,),
        in_specs=[
            pl.BlockSpec((br, D_IN), lambda i: (i, 0)),
            pl.BlockSpec((D_IN, HID), lambda i: (0, 0)),
            pl.BlockSpec((1, HID), lambda i: (0, 0)),
            pl.BlockSpec((br, 1), lambda i: (i, 0)),
        ],
        out_specs=[
            pl.BlockSpec((NQ, br, FW), lambda i: (0, i, 0)),
            pl.BlockSpec((NQ, br, FW), lambda i: (0, i, 0)),
        ],
        out_shape=[
            jax.ShapeDtypeStruct((NQ, NA_PAD, FW), jnp.float32),
            jax.ShapeDtypeStruct((NQ, NA_PAD, FW), jnp.float32),
        ],
    )(x, w, b.reshape(1, HID), d)


def _fc1_paper(x, w, b, d1, d2, lam):
    br = 2000

    def body(x_ref, w_ref, b_ref, d1_ref, d2_ref, lam_ref, xp_ref, ini_ref):
        h = jnp.dot(x_ref[...], w_ref[...],
                    preferred_element_type=jnp.float32) + b_ref[...]
        h = jnp.maximum(h, 0.0)
        u = d1_ref[...] * h
        v = d2_ref[...] * h
        lamm = lam_ref[...]
        for q in range(NQ):
            xp_ref[q] = h[:, q * FW:(q + 1) * FW]
        for hh in range(HOP):
            ini = lamm[hh, 0] * u + lamm[hh, 1] * v
            for q in range(NQ):
                ini_ref[hh, q] = ini[:, q * FW:(q + 1) * FW]

    return pl.pallas_call(
        body,
        grid=(N_P // br,),
        in_specs=[
            pl.BlockSpec((br, D_IN), lambda i: (i, 0)),
            pl.BlockSpec((D_IN, HID), lambda i: (0, 0)),
            pl.BlockSpec((1, HID), lambda i: (0, 0)),
            pl.BlockSpec((br, 1), lambda i: (i, 0)),
            pl.BlockSpec((br, 1), lambda i: (i, 0)),
            pl.BlockSpec((HOP, 2), lambda i: (0, 0)),
        ],
        out_specs=[
            pl.BlockSpec((NQ, br, FW), lambda i: (0, i, 0)),
            pl.BlockSpec((HOP, NQ, br, FW), lambda i: (0, 0, i, 0)),
        ],
        out_shape=[
            jax.ShapeDtypeStruct((NQ, NP_PAD, FW), jnp.float32),
            jax.ShapeDtypeStruct((HOP, NQ, NP_PAD, FW), jnp.float32),
        ],
    )(x, w, b.reshape(1, HID), d1, d2, lam)


def _fc2(hq, w, b):
    br = 2000

    def body(h_ref, w_ref, b_ref, o_ref):
        h = jnp.concatenate([h_ref[q] for q in range(NQ)], axis=1)
        o_ref[...] = jnp.dot(h, w_ref[...],
                             preferred_element_type=jnp.float32) + b_ref[...]

    return pl.pallas_call(
        body,
        grid=(N_P // br,),
        in_specs=[
            pl.BlockSpec((NQ, br, FW), lambda i: (0, i, 0)),
            pl.BlockSpec((HID, OUT), lambda i: (0, 0)),
            pl.BlockSpec((1, OUT), lambda i: (0, 0)),
        ],
        out_specs=pl.BlockSpec((br, OUT), lambda i: (i, 0)),
        out_shape=jax.ShapeDtypeStruct((N_P, OUT), jnp.float32),
    )(hq, w, b.reshape(1, OUT))


def _pad_edges(src, dst, val, tgt):
    npad = tgt - E
    sp = (jnp.arange(npad, dtype=jnp.int32) % 61)
    dp = (jnp.arange(npad, dtype=jnp.int32) % 53)
    vp = jnp.zeros((npad,), jnp.float32)
    return (jnp.concatenate([src.astype(jnp.int32), sp]),
            jnp.concatenate([dst.astype(jnp.int32), dp]),
            jnp.concatenate([val, vp]))


def kernel(x_author, x_paper, dst_ap, src_ap, val_ap, dst_pa, src_pa,
           val_pa, dst_pp, src_pp, val_pp, d_ap, d_pa, d_pp, W1_a, b1_a,
           W1_p, b1_p, W2, b2, lw):
    lam_p = jax.nn.softmax(lw[:, 1:3], axis=-1)
    lam16 = jnp.broadcast_to(lam_p[:, :, None], (HOP, 2, 16))

    s_ap, d_ap_e, v_ap = _pad_edges(src_ap, dst_ap, val_ap, E_PAD_A)
    s_pa, d_pa_e, v_pa = _pad_edges(src_pa, dst_pa, val_pa, E_PAD_P)
    s_pp, d_pp_e, v_pp = _pad_edges(src_pp, dst_pp, val_pp, E_PAD_P)
    s_p = jnp.concatenate([s_pa, s_pp])
    d_p = jnp.concatenate([d_pa_e, d_pp_e])
    v_p = jnp.concatenate([v_pa, v_pp])

    xa_q, base_a = _fc1_author(x_author, W1_a, b1_a, d_ap)
    xp_q, init_p = _fc1_paper(x_paper, W1_p, b1_p, d_pa, d_pp, lam_p)

    base_a = base_a.reshape(NQ * NA_PAD, FW)
    init_p = init_p.reshape(HOP, NQ * NP_PAD, FW)

    hp = xp_q.reshape(NQ * NP_PAD, FW)
    for i in range(HOP):
        ha = _spmm_a(hp, s_ap, d_ap_e, v_ap, base_a)
        hp = _spmm_p(ha, hp, s_p, d_p, v_p, lam16[i, 0], lam16[i, 1],
                     init_p[i])
    return _fc2(hp.reshape(NQ, NP_PAD, FW), W2, b2)


# speedup vs baseline: 4.8775x; 1.0827x over previous
import jax
import jax.numpy as jnp
from jax import lax
from jax.experimental import pallas as pl
from jax.experimental.pallas import tpu as pltpu
from jax.experimental.pallas import tpu_sc as plsc

N_A = 50000
N_P = 100000
D_IN = 128
HID = 64
OUT = 16
HOP = 3
E = 500000

FW = 16
NQ = HID // FW
NC = 2
NS = 16
G = 384
E_PAD_P = 503808
WPT_P = 2 * E_PAD_P // (NS * G)
E_PAD_A = 516096
WPT_A = E_PAD_A // (NS * G)
WB_P = E_PAD_P // G
NA_PAD = 50048
NP_PAD = 100352


def _make_spmm(two_tables, n_out_pad, pad1, pad2, wpt, wb):
    rpt = n_out_pad // NS
    q_per_core = NQ // NC

    def body(*refs):
        if two_tables:
            (t1_ref, t2_ref, pk_ref, lam1_ref, lam2_ref,
             init_ref, out_ref, idxb, sdst, gidx, rowbuf,
             lamv, acc, si0, si1, sg0, sg1, ss0, ss1) = refs
        else:
            (t1_ref, pk_ref, init_ref, out_ref,
             idxb, sdst, gidx, rowbuf,
             lamv, acc, si0, si1, sg0, sg1, ss0, ss1) = refs
            t2_ref = None
        s_idx, s_gat, s_sca = (si0, si1), (sg0, sg1), (ss0, ss1)

        cid = lax.axis_index("c")
        sid = lax.axis_index("s")

        if two_tables:
            pltpu.sync_copy(lam1_ref, lamv.at[0])
            pltpu.sync_copy(lam2_ref, lamv.at[1])

        e16 = [jnp.full((16,), e, jnp.int32) for e in range(16)]

        def fire_idx(t, b):
            tc = jnp.minimum(t, wpt - 1)
            w = sid + tc * NS
            pltpu.async_copy(pk_ref.at[w], idxb.at[b], s_idx[b])

        def wait_idx(b):
            pltpu.make_async_copy(pk_ref.at[0], idxb.at[b],
                                  s_idx[b]).wait()

        def wait_gather(b):
            pltpu.make_async_copy(t1_ref.at[gidx.at[b]],
                                  rowbuf.at[b], s_gat[b]).wait()

        def wait_scatter(b):
            pltpu.make_async_copy(rowbuf.at[b], acc.at[sdst.at[b]],
                                  s_sca[b]).wait()

        for qi in range(q_per_core):
            q = cid * q_per_core + qi
            off_r = sid * rpt
            pltpu.sync_copy(init_ref.at[pl.ds(q * n_out_pad + off_r, rpt)],
                            acc.at[pl.ds(off_r, rpt)])
            plsc.subcore_barrier()

            qb1 = jnp.full((16,), q * pad1, jnp.int32)
            if two_tables:
                qb2 = jnp.full((16,), q * pad2, jnp.int32)

            def comp_gidx_fire(t, b):
                tc = jnp.minimum(t, wpt - 1)
                w = sid + tc * NS
                if two_tables:
                    is2 = w >= wb
                    qb = jnp.where(is2, qb2, qb1)
                else:
                    qb = qb1
                for gs in range(G // 16):
                    sl = pl.ds(gs * 16, 16)
                    gidx[b, sl] = idxb[b, 0, sl] + qb
                if two_tables:
                    @pl.when(is2)
                    def _():
                        pltpu.async_copy(t2_ref.at[gidx.at[b]],
                                         rowbuf.at[b], s_gat[b])

                    @pl.when(jnp.logical_not(is2))
                    def _():
                        pltpu.async_copy(t1_ref.at[gidx.at[b]],
                                         rowbuf.at[b], s_gat[b])
                else:
                    pltpu.async_copy(t1_ref.at[gidx.at[b]],
                                     rowbuf.at[b], s_gat[b])

            def emit_half(t, b):
                nb = 1 - b
                wait_idx(nb)
                comp_gidx_fire(t + 1, nb)
                wait_gather(b)

                @pl.when(t > 0)
                def _():
                    wait_scatter(nb)

                if two_tables:
                    w = sid + t * NS
                    lsel = (w >= wb).astype(jnp.float32)
                    lamvec = (lamv[1, pl.ds(0, 16)] * lsel
                              + lamv[0, pl.ds(0, 16)] * (1.0 - lsel))
                else:
                    lamvec = None
                for gs in range(G // 16):
                    sl = pl.ds(gs * 16, 16)
                    sdst[b, sl] = idxb[b, 1, sl]
                    vals16 = plsc.bitcast(idxb[b, 2, sl], jnp.float32)
                    if lamvec is not None:
                        vals16 = vals16 * lamvec
                    for e in range(16):
                        vb = jnp.take(vals16, e16[e])
                        r = gs * 16 + e
                        rowbuf[b, r, pl.ds(0, 16)] = (
                            rowbuf[b, r, pl.ds(0, 16)] * vb)
                pltpu.async_copy(rowbuf.at[b], acc.at[sdst.at[b]],
                                 s_sca[b], add=True)
                fire_idx(t + 2, b)

            fire_idx(jnp.int32(0), 0)
            fire_idx(jnp.int32(1), 1)
            wait_idx(0)
            comp_gidx_fire(jnp.int32(0), 0)

            def pair(tt, _):
                emit_half(tt * 2, 0)
                emit_half(tt * 2 + 1, 1)
                return 0

            lax.fori_loop(0, wpt // 2, pair, 0)

            wait_idx(1)
            wait_gather(0)
            wait_scatter(1)

            plsc.subcore_barrier()
            pltpu.sync_copy(acc.at[pl.ds(off_r, rpt)],
                            out_ref.at[pl.ds(q * n_out_pad + off_r, rpt)])
            if qi + 1 < q_per_core:
                plsc.subcore_barrier()

    mesh = plsc.VectorSubcoreMesh(core_axis_name="c", subcore_axis_name="s")
    scratch = [
        pltpu.VMEM((2, 3, G), jnp.int32),
        pltpu.VMEM((2, G), jnp.int32),
        pltpu.VMEM((2, G), jnp.int32),
        pltpu.VMEM((2, G, FW), jnp.float32),
        pltpu.VMEM((2, 16), jnp.float32),
        pltpu.VMEM_SHARED((n_out_pad, FW), jnp.float32),
        pltpu.SemaphoreType.DMA,
        pltpu.SemaphoreType.DMA,
        pltpu.SemaphoreType.DMA,
        pltpu.SemaphoreType.DMA,
        pltpu.SemaphoreType.DMA,
        pltpu.SemaphoreType.DMA,
    ]
    return pl.kernel(
        body,
        out_type=jax.ShapeDtypeStruct((NQ * n_out_pad, FW), jnp.float32),
        mesh=mesh,
        scratch_types=scratch,
        compiler_params=pltpu.CompilerParams(use_tc_tiling_on_sc=False,
                                             needs_layout_passes=False),
    )


_spmm_a = _make_spmm(two_tables=False, n_out_pad=NA_PAD, pad1=NP_PAD,
                     pad2=0, wpt=WPT_A, wb=0)
_spmm_p = _make_spmm(two_tables=True, n_out_pad=NP_PAD, pad1=NA_PAD,
                     pad2=NP_PAD, wpt=WPT_P, wb=WB_P)


def _fc1_author(x, w, b, d):
    br = 2000

    def body(x_ref, w_ref, b_ref, d_ref, xa_ref, ba_ref):
        h = jnp.dot(x_ref[...], w_ref[...],
                    preferred_element_type=jnp.float32) + b_ref[...]
        h = jnp.maximum(h, 0.0)
        dh = d_ref[...] * h
        for q in range(NQ):
            xa_ref[q] = h[:, q * FW:(q + 1) * FW]
            ba_ref[q] = dh[:, q * FW:(q + 1) * FW]

    return pl.pallas_call(
        body,
        grid=(N_A // br,),
        in_specs=[
            pl.BlockSpec((br, D_IN), lambda i: (i, 0)),
            pl.BlockSpec((D_IN, HID), lambda i: (0, 0)),
            pl.BlockSpec((1, HID), lambda i: (0, 0)),
            pl.BlockSpec((br, 1), lambda i: (i, 0)),
        ],
        out_specs=[
            pl.BlockSpec((NQ, br, FW), lambda i: (0, i, 0)),
            pl.BlockSpec((NQ, br, FW), lambda i: (0, i, 0)),
        ],
        out_shape=[
            jax.ShapeDtypeStruct((NQ, NA_PAD, FW), jnp.float32),
            jax.ShapeDtypeStruct((NQ, NA_PAD, FW), jnp.float32),
        ],
    )(x, w, b.reshape(1, HID), d)


def _fc1_paper(x, w, b, d1, d2, lam):
    br = 2000

    def body(x_ref, w_ref, b_ref, d1_ref, d2_ref, lam_ref, xp_ref, ini_ref):
        h = jnp.dot(x_ref[...], w_ref[...],
                    preferred_element_type=jnp.float32) + b_ref[...]
        h = jnp.maximum(h, 0.0)
        u = d1_ref[...] * h
        v = d2_ref[...] * h
        lamm = lam_ref[...]
        for q in range(NQ):
            xp_ref[q] = h[:, q * FW:(q + 1) * FW]
        for hh in range(HOP):
            ini = lamm[hh, 0] * u + lamm[hh, 1] * v
            for q in range(NQ):
                ini_ref[hh, q] = ini[:, q * FW:(q + 1) * FW]

    return pl.pallas_call(
        body,
        grid=(N_P // br,),
        in_specs=[
            pl.BlockSpec((br, D_IN), lambda i: (i, 0)),
            pl.BlockSpec((D_IN, HID), lambda i: (0, 0)),
            pl.BlockSpec((1, HID), lambda i: (0, 0)),
            pl.BlockSpec((br, 1), lambda i: (i, 0)),
            pl.BlockSpec((br, 1), lambda i: (i, 0)),
            pl.BlockSpec((HOP, 2), lambda i: (0, 0)),
        ],
        out_specs=[
            pl.BlockSpec((NQ, br, FW), lambda i: (0, i, 0)),
            pl.BlockSpec((HOP, NQ, br, FW), lambda i: (0, 0, i, 0)),
        ],
        out_shape=[
            jax.ShapeDtypeStruct((NQ, NP_PAD, FW), jnp.float32),
            jax.ShapeDtypeStruct((HOP, NQ, NP_PAD, FW), jnp.float32),
        ],
    )(x, w, b.reshape(1, HID), d1, d2, lam)


def _fc2(hq, w, b):
    br = 2000

    def body(h_ref, w_ref, b_ref, o_ref):
        h = jnp.concatenate([h_ref[q] for q in range(NQ)], axis=1)
        o_ref[...] = jnp.dot(h, w_ref[...],
                             preferred_element_type=jnp.float32) + b_ref[...]

    return pl.pallas_call(
        body,
        grid=(N_P // br,),
        in_specs=[
            pl.BlockSpec((NQ, br, FW), lambda i: (0, i, 0)),
            pl.BlockSpec((HID, OUT), lambda i: (0, 0)),
            pl.BlockSpec((1, OUT), lambda i: (0, 0)),
        ],
        out_specs=pl.BlockSpec((br, OUT), lambda i: (i, 0)),
        out_shape=jax.ShapeDtypeStruct((N_P, OUT), jnp.float32),
    )(hq, w, b.reshape(1, OUT))


def _pack_edges(src, dst, val, tgt):
    npad = tgt - E
    sp = (jnp.arange(npad, dtype=jnp.int32) % 61)
    dp = (jnp.arange(npad, dtype=jnp.int32) % 53)
    vp = jnp.zeros((npad,), jnp.float32)
    s = jnp.concatenate([src.astype(jnp.int32), sp]).reshape(-1, G)
    d = jnp.concatenate([dst.astype(jnp.int32), dp]).reshape(-1, G)
    v = jax.lax.bitcast_convert_type(
        jnp.concatenate([val, vp]), jnp.int32).reshape(-1, G)
    return jnp.stack([s, d, v], axis=1)


def kernel(x_author, x_paper, dst_ap, src_ap, val_ap, dst_pa, src_pa,
           val_pa, dst_pp, src_pp, val_pp, d_ap, d_pa, d_pp, W1_a, b1_a,
           W1_p, b1_p, W2, b2, lw):
    lam_p = jax.nn.softmax(lw[:, 1:3], axis=-1)
    lam16 = jnp.broadcast_to(lam_p[:, :, None], (HOP, 2, 16))

    pk_a = _pack_edges(src_ap, dst_ap, val_ap, E_PAD_A)
    pk_p = jnp.concatenate([_pack_edges(src_pa, dst_pa, val_pa, E_PAD_P),
                            _pack_edges(src_pp, dst_pp, val_pp, E_PAD_P)])

    xa_q, base_a = _fc1_author(x_author, W1_a, b1_a, d_ap)
    xp_q, init_p = _fc1_paper(x_paper, W1_p, b1_p, d_pa, d_pp, lam_p)

    base_a = base_a.reshape(NQ * NA_PAD, FW)
    init_p = init_p.reshape(HOP, NQ * NP_PAD, FW)

    hp = xp_q.reshape(NQ * NP_PAD, FW)
    for i in range(HOP):
        ha = _spmm_a(hp, pk_a, base_a)
        hp = _spmm_p(ha, hp, pk_p, lam16[i, 0], lam16[i, 1], init_p[i])
    return _fc2(hp.reshape(NQ, NP_PAD, FW), W2, b2)
